# baseline (speedup 1.0000x reference)
import functools

import jax
import jax.numpy as jnp
from jax import lax
from jax.experimental import pallas as pl
from jax.experimental.pallas import tpu as pltpu
from jax.experimental.pallas import tpu_sc as plsc

_B, _L, _V = 32, 4, 100000
_INVALID = -1
_NBUF = 3


def _scan_body(dt_ref, tp_hbm, dp_hbm, q_hbm, rec_ref, dpat_ref, tpat_ref,
               tp_buf, dp_buf, q_buf, tp_sem, dp_sem, q_sem):
    b = pl.program_id(0)
    n = pl.num_programs(0)

    def issue(step, slot):
        pltpu.make_async_copy(tp_hbm.at[step, :_L], tp_buf.at[slot],
                              tp_sem.at[slot]).start()
        pltpu.make_async_copy(dp_hbm.at[step], dp_buf.at[slot],
                              tp_sem.at[slot]).start()
        pltpu.make_async_copy(q_hbm.at[step], q_buf.at[slot],
                              tp_sem.at[slot]).start()

    @pl.when(b == 0)
    def _():
        for k in range(_NBUF - 1):
            issue(k, k)

    slot = jax.lax.rem(b, _NBUF)

    @pl.when(b + _NBUF - 1 < n)
    def _():
        issue(b + _NBUF - 1, jax.lax.rem(b + _NBUF - 1, _NBUF))

    pltpu.make_async_copy(tp_hbm.at[b, :_L], tp_buf.at[slot],
                          tp_sem.at[slot]).wait()
    pltpu.make_async_copy(dp_hbm.at[b], dp_buf.at[slot],
                          tp_sem.at[slot]).wait()
    pltpu.make_async_copy(q_hbm.at[b], q_buf.at[slot],
                          tp_sem.at[slot]).wait()

    tpb = tp_buf[slot]
    dpb = dp_buf[slot]
    qb = q_buf[slot]
    ratio = jnp.maximum(tpb - dpb, 0.0) / qb
    m = jnp.max(ratio, axis=1, keepdims=True)
    col = jax.lax.broadcasted_iota(jnp.int32, (_L, _V), 1)
    idx = jnp.min(jnp.where(ratio == m, col, _V), axis=1, keepdims=True)
    rec_ref[0] = idx
    lane = jax.lax.broadcasted_iota(jnp.int32, (1, 128), 1)
    for l in range(_L):
        tok = dt_ref[b, l]
        base = pl.multiple_of((tok // 128) * 128, 128)
        off = tok - base
        dvec = dp_buf[slot, l, pl.ds(base, 128)].reshape(1, 128)
        tvec = tp_buf[slot, l, pl.ds(base, 128)].reshape(1, 128)
        hit = lane == off
        dpat_ref[0, l, :] = jnp.sum(jnp.where(hit, dvec, 0.0), axis=1)
        tpat_ref[0, l, :] = jnp.sum(jnp.where(hit, tvec, 0.0), axis=1)


def _vgather(vec, idx):
    return lax.gather(
        vec, idx[:, None],
        lax.GatherDimensionNumbers(offset_dims=(), collapsed_slice_dims=(0,),
                                   start_index_map=(0,)),
        (1,), mode=lax.GatherScatterMode.PROMISE_IN_BOUNDS)


def _sc_epilogue_body(dpat_hbm, tpat_hbm, u_hbm, dt_hbm, rec_hbm,
                      bonus_hbm, out_hbm, dpat_v, tpat_v, u_v, dt_v, rec_v,
                      acc_v, bonus_v, out_v):
    first = (lax.axis_index("c") == 0) & (lax.axis_index("s") == 0)

    @pl.when(first)
    def _():
        pltpu.sync_copy(dpat_hbm, dpat_v)
        pltpu.sync_copy(tpat_hbm, tpat_v)
        pltpu.sync_copy(u_hbm, u_v)
        pltpu.sync_copy(dt_hbm, dt_v)
        pltpu.sync_copy(rec_hbm, rec_v)
        pltpu.sync_copy(bonus_hbm, bonus_v)
        iota = lax.iota(jnp.int32, 16)
        one = jnp.full((16,), 1, jnp.int32)
        zero = jnp.full((16,), 0, jnp.int32)
        inval = jnp.full((16,), _INVALID, jnp.int32)
        lvec = jnp.full((16,), _L, jnp.int32)
        l1vec = jnp.full((16,), _L + 1, jnp.int32)
        for c in range(_B * _L // 16):
            sl = pl.ds(c * 16, 16)
            a = u_v[sl] * dpat_v[sl] <= tpat_v[sl]
            acc_v[sl] = jnp.where(a, one, zero)
        for c in range(_B * (_L + 1) // 16):
            b0 = (c * 16) // (_L + 1)
            lin = c * 16 + iota
            b = lax.div(lin, l1vec)
            pos = lin - b * l1vec
            relb = b - jnp.full((16,), b0, jnp.int32)
            aw = acc_v[pl.ds(_L * b0, 16)]
            dw = dt_v[pl.ds(_L * b0, 16)]
            rw = rec_v[pl.ds(_L * b0, 16)]
            bw = bonus_v[pl.ds(16 * (b0 // 16), 16)]
            a0 = _vgather(aw, relb * _L)
            a1 = _vgather(aw, relb * _L + 1)
            a2 = _vgather(aw, relb * _L + 2)
            a3 = _vgather(aw, relb * _L + 3)
            nab = a0 * (one + a1 * (one + a2 * (one + a3)))
            dtg = _vgather(dw, jnp.minimum(relb * _L + pos,
                                           jnp.full((16,), 15, jnp.int32)))
            recg = _vgather(
                rw, relb * _L + jnp.minimum(jnp.maximum(nab, zero),
                                            jnp.full((16,), _L - 1,
                                                     jnp.int32)))
            bong = _vgather(bw, b - jnp.full((16,), 16 * (b0 // 16),
                                             jnp.int32))
            rej = jnp.where(nab < lvec, recg, bong)
            o = jnp.where(pos == nab, rej,
                          jnp.where(pos < nab, dtg, inval))
            out_v[pl.ds(c * 16, 16)] = o
        pltpu.sync_copy(out_v, out_hbm)


_sc_epilogue = functools.partial(
    pl.kernel,
    mesh=plsc.VectorSubcoreMesh(core_axis_name="c", subcore_axis_name="s"),
    out_type=jax.ShapeDtypeStruct((_B * (_L + 1),), jnp.int32),
    scratch_types=[
        pltpu.VMEM((_B * _L,), jnp.float32),
        pltpu.VMEM((_B * _L,), jnp.float32),
        pltpu.VMEM((_B * _L,), jnp.float32),
        pltpu.VMEM((_B * _L,), jnp.int32),
        pltpu.VMEM((_B * _L,), jnp.int32),
        pltpu.VMEM((_B * _L,), jnp.int32),
        pltpu.VMEM((_B,), jnp.int32),
        pltpu.VMEM((_B * (_L + 1),), jnp.int32),
    ],
)(_sc_epilogue_body)


def kernel(draft_probs, target_probs, uniform, q, draft_token_ids,
           bonus_token_ids):
    rec, dpat, tpat = pl.pallas_call(
        _scan_body,
        grid=(_B,),
        in_specs=[
            pl.BlockSpec(memory_space=pltpu.SMEM),
            pl.BlockSpec(memory_space=pl.ANY),
            pl.BlockSpec(memory_space=pl.ANY),
            pl.BlockSpec(memory_space=pl.ANY),
        ],
        out_specs=[
            pl.BlockSpec((1, _L, 1), lambda b: (b, 0, 0)),
            pl.BlockSpec((1, _L, 1), lambda b: (b, 0, 0)),
            pl.BlockSpec((1, _L, 1), lambda b: (b, 0, 0)),
        ],
        out_shape=[
            jax.ShapeDtypeStruct((_B, _L, 1), jnp.int32),
            jax.ShapeDtypeStruct((_B, _L, 1), jnp.float32),
            jax.ShapeDtypeStruct((_B, _L, 1), jnp.float32),
        ],
        scratch_shapes=[
            pltpu.VMEM((_NBUF, _L, _V), jnp.float32),
            pltpu.VMEM((_NBUF, _L, _V), jnp.float32),
            pltpu.VMEM((_NBUF, _L, _V), jnp.float32),
            pltpu.SemaphoreType.DMA((_NBUF,)),
            pltpu.SemaphoreType.DMA((_NBUF,)),
            pltpu.SemaphoreType.DMA((_NBUF,)),
        ],
    )(draft_token_ids, target_probs.reshape(_B, _L + 1, _V), draft_probs, q)

    out = _sc_epilogue(dpat.reshape(_B * _L), tpat.reshape(_B * _L),
                       uniform.reshape(_B * _L),
                       draft_token_ids.reshape(_B * _L),
                       rec.reshape(_B * _L), bonus_token_ids.reshape(_B))
    return out.reshape(_B, _L + 1)


# speedup vs baseline: 1.0132x; 1.0132x over previous
import functools

import jax
import jax.numpy as jnp
from jax import lax
from jax.experimental import pallas as pl
from jax.experimental.pallas import tpu as pltpu
from jax.experimental.pallas import tpu_sc as plsc

_B, _L, _V = 32, 4, 100000
_INVALID = -1
_NBUF = 3


def _scan_body(dt_ref, u_ref, tp_hbm, dp_hbm, q_hbm, rec_ref, acc_ref,
               tp_buf, dp_buf, q_buf, tp_sem, dp_sem, q_sem):
    b = pl.program_id(0)
    n = pl.num_programs(0)

    def issue(step, slot):
        pltpu.make_async_copy(tp_hbm.at[step, :_L], tp_buf.at[slot],
                              tp_sem.at[slot]).start()
        pltpu.make_async_copy(dp_hbm.at[step], dp_buf.at[slot],
                              tp_sem.at[slot]).start()
        pltpu.make_async_copy(q_hbm.at[step], q_buf.at[slot],
                              tp_sem.at[slot]).start()

    @pl.when(b == 0)
    def _():
        for k in range(_NBUF - 1):
            issue(k, k)

    slot = jax.lax.rem(b, _NBUF)

    @pl.when(b + _NBUF - 1 < n)
    def _():
        issue(b + _NBUF - 1, jax.lax.rem(b + _NBUF - 1, _NBUF))

    pltpu.make_async_copy(tp_hbm.at[b, :_L], tp_buf.at[slot],
                          tp_sem.at[slot]).wait()
    pltpu.make_async_copy(dp_hbm.at[b], dp_buf.at[slot],
                          tp_sem.at[slot]).wait()
    pltpu.make_async_copy(q_hbm.at[b], q_buf.at[slot],
                          tp_sem.at[slot]).wait()

    tpb = tp_buf[slot]
    dpb = dp_buf[slot]
    qb = q_buf[slot]
    ratio = jnp.maximum(tpb - dpb, 0.0) / qb
    m = jnp.max(ratio, axis=1, keepdims=True)
    col = jax.lax.broadcasted_iota(jnp.int32, (_L, _V), 1)
    idx = jnp.min(jnp.where(ratio == m, col, _V), axis=1, keepdims=True)
    rec_ref[0] = idx
    lane = jax.lax.broadcasted_iota(jnp.int32, (1, 128), 1)
    for l in range(_L):
        tok = dt_ref[b, l]
        base = pl.multiple_of((tok // 128) * 128, 128)
        off = tok - base
        dvec = dp_buf[slot, l, pl.ds(base, 128)].reshape(1, 128)
        tvec = tp_buf[slot, l, pl.ds(base, 128)].reshape(1, 128)
        hit = lane == off
        dpat = jnp.sum(jnp.where(hit, dvec, 0.0), axis=1)
        tpat = jnp.sum(jnp.where(hit, tvec, 0.0), axis=1)
        acc_ref[0, l, :] = jnp.where(u_ref[b, l] * dpat <= tpat, 1,
                                     0).astype(jnp.int32)


def _vgather(vec, idx):
    return lax.gather(
        vec, idx[:, None],
        lax.GatherDimensionNumbers(offset_dims=(), collapsed_slice_dims=(0,),
                                   start_index_map=(0,)),
        (1,), mode=lax.GatherScatterMode.PROMISE_IN_BOUNDS)


def _sc_epilogue_body(acc_hbm, dt_hbm, rec_hbm, bonus_hbm, out_hbm,
                      dt_v, rec_v, acc_v, bonus_v, out_v):
    first = (lax.axis_index("c") == 0) & (lax.axis_index("s") == 0)

    @pl.when(first)
    def _():
        pltpu.sync_copy(acc_hbm, acc_v)
        pltpu.sync_copy(dt_hbm, dt_v)
        pltpu.sync_copy(rec_hbm, rec_v)
        pltpu.sync_copy(bonus_hbm, bonus_v)
        iota = lax.iota(jnp.int32, 16)
        one = jnp.full((16,), 1, jnp.int32)
        zero = jnp.full((16,), 0, jnp.int32)
        inval = jnp.full((16,), _INVALID, jnp.int32)
        lvec = jnp.full((16,), _L, jnp.int32)
        l1vec = jnp.full((16,), _L + 1, jnp.int32)
        for c in range(_B * (_L + 1) // 16):
            b0 = (c * 16) // (_L + 1)
            lin = c * 16 + iota
            b = lax.div(lin, l1vec)
            pos = lin - b * l1vec
            relb = b - jnp.full((16,), b0, jnp.int32)
            aw = acc_v[pl.ds(_L * b0, 16)]
            dw = dt_v[pl.ds(_L * b0, 16)]
            rw = rec_v[pl.ds(_L * b0, 16)]
            bw = bonus_v[pl.ds(16 * (b0 // 16), 16)]
            a0 = _vgather(aw, relb * _L)
            a1 = _vgather(aw, relb * _L + 1)
            a2 = _vgather(aw, relb * _L + 2)
            a3 = _vgather(aw, relb * _L + 3)
            nab = a0 * (one + a1 * (one + a2 * (one + a3)))
            dtg = _vgather(dw, jnp.minimum(relb * _L + pos,
                                           jnp.full((16,), 15, jnp.int32)))
            recg = _vgather(
                rw, relb * _L + jnp.minimum(jnp.maximum(nab, zero),
                                            jnp.full((16,), _L - 1,
                                                     jnp.int32)))
            bong = _vgather(bw, b - jnp.full((16,), 16 * (b0 // 16),
                                             jnp.int32))
            rej = jnp.where(nab < lvec, recg, bong)
            o = jnp.where(pos == nab, rej,
                          jnp.where(pos < nab, dtg, inval))
            out_v[pl.ds(c * 16, 16)] = o
        pltpu.sync_copy(out_v, out_hbm)


_sc_epilogue = functools.partial(
    pl.kernel,
    mesh=plsc.VectorSubcoreMesh(core_axis_name="c", subcore_axis_name="s"),
    out_type=jax.ShapeDtypeStruct((_B * (_L + 1),), jnp.int32),
    scratch_types=[
        pltpu.VMEM((_B * _L,), jnp.int32),
        pltpu.VMEM((_B * _L,), jnp.int32),
        pltpu.VMEM((_B * _L,), jnp.int32),
        pltpu.VMEM((_B,), jnp.int32),
        pltpu.VMEM((_B * (_L + 1),), jnp.int32),
    ],
)(_sc_epilogue_body)


def kernel(draft_probs, target_probs, uniform, q, draft_token_ids,
           bonus_token_ids):
    rec, acc = pl.pallas_call(
        _scan_body,
        grid=(_B,),
        in_specs=[
            pl.BlockSpec(memory_space=pltpu.SMEM),
            pl.BlockSpec(memory_space=pltpu.SMEM),
            pl.BlockSpec(memory_space=pl.ANY),
            pl.BlockSpec(memory_space=pl.ANY),
            pl.BlockSpec(memory_space=pl.ANY),
        ],
        out_specs=[
            pl.BlockSpec((1, _L, 1), lambda b: (b, 0, 0)),
            pl.BlockSpec((1, _L, 1), lambda b: (b, 0, 0)),
        ],
        out_shape=[
            jax.ShapeDtypeStruct((_B, _L, 1), jnp.int32),
            jax.ShapeDtypeStruct((_B, _L, 1), jnp.int32),
        ],
        scratch_shapes=[
            pltpu.VMEM((_NBUF, _L, _V), jnp.float32),
            pltpu.VMEM((_NBUF, _L, _V), jnp.float32),
            pltpu.VMEM((_NBUF, _L, _V), jnp.float32),
            pltpu.SemaphoreType.DMA((_NBUF,)),
            pltpu.SemaphoreType.DMA((_NBUF,)),
            pltpu.SemaphoreType.DMA((_NBUF,)),
        ],
    )(draft_token_ids, uniform, target_probs.reshape(_B, _L + 1, _V),
      draft_probs, q)

    out = _sc_epilogue(acc.reshape(_B * _L),
                       draft_token_ids.reshape(_B * _L),
                       rec.reshape(_B * _L), bonus_token_ids.reshape(_B))
    return out.reshape(_B, _L + 1)


# speedup vs baseline: 1.0150x; 1.0017x over previous
import functools

import jax
import jax.numpy as jnp
from jax import lax
from jax.experimental import pallas as pl
from jax.experimental.pallas import tpu as pltpu
from jax.experimental.pallas import tpu_sc as plsc

_B, _L, _V = 32, 4, 100000
_INVALID = -1
_NBUF = 3


def _scan_body(dt_ref, u_ref, tp_hbm, dp_hbm, q_hbm, rec_ref, acc_ref,
               tp_buf, dp_buf, q_buf, sem):
    b = pl.program_id(0)
    n = pl.num_programs(0)

    def issue(step, slot):
        pltpu.make_async_copy(tp_hbm.at[step, :_L], tp_buf.at[slot],
                              sem.at[slot]).start()
        pltpu.make_async_copy(dp_hbm.at[step], dp_buf.at[slot],
                              sem.at[slot]).start()
        pltpu.make_async_copy(q_hbm.at[step], q_buf.at[slot],
                              sem.at[slot]).start()

    @pl.when(b == 0)
    def _():
        for k in range(_NBUF - 1):
            issue(k, k)

    slot = jax.lax.rem(b, _NBUF)

    @pl.when(b + _NBUF - 1 < n)
    def _():
        issue(b + _NBUF - 1, jax.lax.rem(b + _NBUF - 1, _NBUF))

    pltpu.make_async_copy(tp_hbm.at[b, :_L], tp_buf.at[slot],
                          sem.at[slot]).wait()
    pltpu.make_async_copy(dp_hbm.at[b], dp_buf.at[slot],
                          sem.at[slot]).wait()
    pltpu.make_async_copy(q_hbm.at[b], q_buf.at[slot],
                          sem.at[slot]).wait()

    tpb = tp_buf[slot]
    dpb = dp_buf[slot]
    qb = q_buf[slot]
    ratio = jnp.maximum(tpb - dpb, 0.0) / qb
    m = jnp.max(ratio, axis=1, keepdims=True)
    col = jax.lax.broadcasted_iota(jnp.int32, (_L, _V), 1)
    idx = jnp.min(jnp.where(ratio == m, col, _V), axis=1, keepdims=True)
    rec_ref[0] = idx
    lane = jax.lax.broadcasted_iota(jnp.int32, (1, 128), 1)
    for l in range(_L):
        tok = dt_ref[b, l]
        base = pl.multiple_of((tok // 128) * 128, 128)
        off = tok - base
        dvec = dp_buf[slot, l, pl.ds(base, 128)].reshape(1, 128)
        tvec = tp_buf[slot, l, pl.ds(base, 128)].reshape(1, 128)
        hit = lane == off
        dpat = jnp.sum(jnp.where(hit, dvec, 0.0), axis=1)
        tpat = jnp.sum(jnp.where(hit, tvec, 0.0), axis=1)
        acc_ref[0, l, :] = jnp.where(u_ref[b, l] * dpat <= tpat, 1,
                                     0).astype(jnp.int32)


def _vgather(vec, idx):
    return lax.gather(
        vec, idx[:, None],
        lax.GatherDimensionNumbers(offset_dims=(), collapsed_slice_dims=(0,),
                                   start_index_map=(0,)),
        (1,), mode=lax.GatherScatterMode.PROMISE_IN_BOUNDS)


def _sc_epilogue_body(acc_hbm, dt_hbm, rec_hbm, bonus_hbm, out_hbm,
                      dt_v, rec_v, acc_v, bonus_v, out_v):
    first = (lax.axis_index("c") == 0) & (lax.axis_index("s") == 0)

    @pl.when(first)
    def _():
        pltpu.sync_copy(acc_hbm, acc_v)
        pltpu.sync_copy(dt_hbm, dt_v)
        pltpu.sync_copy(rec_hbm, rec_v)
        pltpu.sync_copy(bonus_hbm, bonus_v)
        iota = lax.iota(jnp.int32, 16)
        one = jnp.full((16,), 1, jnp.int32)
        zero = jnp.full((16,), 0, jnp.int32)
        inval = jnp.full((16,), _INVALID, jnp.int32)
        lvec = jnp.full((16,), _L, jnp.int32)
        l1vec = jnp.full((16,), _L + 1, jnp.int32)
        for c in range(_B * (_L + 1) // 16):
            b0 = (c * 16) // (_L + 1)
            lin = c * 16 + iota
            b = lax.div(lin, l1vec)
            pos = lin - b * l1vec
            relb = b - jnp.full((16,), b0, jnp.int32)
            aw = acc_v[pl.ds(_L * b0, 16)]
            dw = dt_v[pl.ds(_L * b0, 16)]
            rw = rec_v[pl.ds(_L * b0, 16)]
            bw = bonus_v[pl.ds(16 * (b0 // 16), 16)]
            a0 = _vgather(aw, relb * _L)
            a1 = _vgather(aw, relb * _L + 1)
            a2 = _vgather(aw, relb * _L + 2)
            a3 = _vgather(aw, relb * _L + 3)
            nab = a0 * (one + a1 * (one + a2 * (one + a3)))
            dtg = _vgather(dw, jnp.minimum(relb * _L + pos,
                                           jnp.full((16,), 15, jnp.int32)))
            recg = _vgather(
                rw, relb * _L + jnp.minimum(jnp.maximum(nab, zero),
                                            jnp.full((16,), _L - 1,
                                                     jnp.int32)))
            bong = _vgather(bw, b - jnp.full((16,), 16 * (b0 // 16),
                                             jnp.int32))
            rej = jnp.where(nab < lvec, recg, bong)
            o = jnp.where(pos == nab, rej,
                          jnp.where(pos < nab, dtg, inval))
            out_v[pl.ds(c * 16, 16)] = o
        pltpu.sync_copy(out_v, out_hbm)


_sc_epilogue = functools.partial(
    pl.kernel,
    mesh=plsc.VectorSubcoreMesh(core_axis_name="c", subcore_axis_name="s"),
    out_type=jax.ShapeDtypeStruct((_B * (_L + 1),), jnp.int32),
    scratch_types=[
        pltpu.VMEM((_B * _L,), jnp.int32),
        pltpu.VMEM((_B * _L,), jnp.int32),
        pltpu.VMEM((_B * _L,), jnp.int32),
        pltpu.VMEM((_B,), jnp.int32),
        pltpu.VMEM((_B * (_L + 1),), jnp.int32),
    ],
)(_sc_epilogue_body)


def kernel(draft_probs, target_probs, uniform, q, draft_token_ids,
           bonus_token_ids):
    rec, acc = pl.pallas_call(
        _scan_body,
        grid=(_B,),
        in_specs=[
            pl.BlockSpec(memory_space=pltpu.SMEM),
            pl.BlockSpec(memory_space=pltpu.SMEM),
            pl.BlockSpec(memory_space=pl.ANY),
            pl.BlockSpec(memory_space=pl.ANY),
            pl.BlockSpec(memory_space=pl.ANY),
        ],
        out_specs=[
            pl.BlockSpec((1, _L, 1), lambda b: (b, 0, 0)),
            pl.BlockSpec((1, _L, 1), lambda b: (b, 0, 0)),
        ],
        out_shape=[
            jax.ShapeDtypeStruct((_B, _L, 1), jnp.int32),
            jax.ShapeDtypeStruct((_B, _L, 1), jnp.int32),
        ],
        scratch_shapes=[
            pltpu.VMEM((_NBUF, _L, _V), jnp.float32),
            pltpu.VMEM((_NBUF, _L, _V), jnp.float32),
            pltpu.VMEM((_NBUF, _L, _V), jnp.float32),
            pltpu.SemaphoreType.DMA((_NBUF,)),
        ],
    )(draft_token_ids, uniform, target_probs.reshape(_B, _L + 1, _V),
      draft_probs, q)

    out = _sc_epilogue(acc.reshape(_B * _L),
                       draft_token_ids.reshape(_B * _L),
                       rec.reshape(_B * _L), bonus_token_ids.reshape(_B))
    return out.reshape(_B, _L + 1)
